# baseline (speedup 1.0000x reference)
import jax
import jax.numpy as jnp
from jax import lax
from jax.experimental import pallas as pl
from jax.experimental.pallas import tpu as pltpu
from jax.experimental.pallas import tpu_sc as plsc

BATCH = 128
SEQ_LEN = 32768
NUMEL = 2048

LANES = 16
NUM_CORES = 2
NUM_SUBCORES = 16
NUM_WORKERS = NUM_CORES * NUM_SUBCORES
ROWS_PER_WORKER = BATCH // NUM_WORKERS

BS = 128
BPB = BS * LANES
NBLK = SEQ_LEN // BPB

IB = 128
KPB = IB // LANES
NB = BPB // IB

OUT_PAD = BATCH * NUMEL


def _body(mask_hbm, out_hbm, mbuf, rbuf, idxbuf, valbuf, sem):
    wid = lax.axis_index("s") * NUM_CORES + lax.axis_index("c")
    iota = lax.broadcasted_iota(jnp.int32, (LANES,), 0)
    iota_bs = iota * BS
    zeros16 = jnp.zeros((LANES,), jnp.int32)
    dump = OUT_PAD + wid

    for rl in range(ROWS_PER_WORKER):
        row = wid * ROWS_PER_WORKER + rl
        mrow = pl.multiple_of(row * SEQ_LEN, SEQ_LEN)
        orow = row * NUMEL

        def pass_a(k, c):
            vals = mbuf[pl.ds(k * LANES, LANES)]
            rbuf[pl.ds(k * LANES, LANES)] = c
            return c + vals

        def blk_body(blk, t):
            active = t < NUMEL

            @pl.when(active)
            def _load():
                off = pl.multiple_of(mrow + blk * BPB, BPB)
                pltpu.sync_copy(mask_hbm.at[pl.ds(off, BPB)], mbuf)

            n = jnp.where(active, BS, 0)
            c = lax.fori_loop(0, n, pass_a, zeros16)

            bv = zeros16
            run_t = t
            for j in range(LANES):
                bv = jnp.where(iota == j, run_t, bv)
                run_t = run_t + c[j]

            blk_base = blk * BPB

            def pass_b(k, d):
                vals = mbuf[pl.ds(k * LANES, LANES)]
                r = rbuf[pl.ds(k * LANES, LANES)]
                rt = bv + r
                valid = (vals != 0) & (rt < NUMEL)
                idxbuf[k // KPB, pl.ds((k % KPB) * LANES, LANES)] = (
                    jnp.where(valid, orow + rt, dump))
                valbuf[k // KPB, pl.ds((k % KPB) * LANES, LANES)] = (
                    blk_base + iota_bs + k)
                return d

            lax.fori_loop(0, n, pass_b, jnp.int32(0))

            @pl.when(active)
            def _scatter():
                for b in range(NB):
                    pltpu.async_copy(valbuf.at[b],
                                     out_hbm.at[idxbuf.at[b]], sem).wait()

            return jnp.where(active, run_t, t)

        t = lax.fori_loop(0, NBLK, blk_body, jnp.int32(0))

        @pl.when(t < NUMEL)
        def _false_sweep():
            need = NUMEL - t

            def blk_body_f(blk, f):
                active = f < need

                @pl.when(active)
                def _load():
                    off = pl.multiple_of(mrow + blk * BPB, BPB)
                    pltpu.sync_copy(mask_hbm.at[pl.ds(off, BPB)], mbuf)

                n = jnp.where(active, BS, 0)
                c = lax.fori_loop(0, n, pass_a, zeros16)

                bfv = zeros16
                run_f = f
                for j in range(LANES):
                    bfv = jnp.where(iota == j, run_f, bfv)
                    run_f = run_f + (BS - c[j])

                blk_base = blk * BPB

                def pass_bf(k, d):
                    vals = mbuf[pl.ds(k * LANES, LANES)]
                    r = rbuf[pl.ds(k * LANES, LANES)]
                    rf = bfv + (k - r)
                    valid = (vals == 0) & (rf < need)
                    idxbuf[k // KPB, pl.ds((k % KPB) * LANES, LANES)] = (
                        jnp.where(valid, orow + t + rf, dump))
                    valbuf[k // KPB, pl.ds((k % KPB) * LANES, LANES)] = (
                        blk_base + iota_bs + k)
                    return d

                lax.fori_loop(0, n, pass_bf, jnp.int32(0))

                @pl.when(active)
                def _scatter():
                    for b in range(NB):
                        pltpu.async_copy(valbuf.at[b],
                                         out_hbm.at[idxbuf.at[b]],
                                         sem).wait()

                return jnp.where(active, run_f, f)

            lax.fori_loop(0, NBLK, blk_body_f, jnp.int32(0))


_sc_call = pl.kernel(
    _body,
    out_type=jax.ShapeDtypeStruct((BATCH * NUMEL + NUM_WORKERS,), jnp.int32),
    mesh=plsc.VectorSubcoreMesh(
        core_axis_name="c", subcore_axis_name="s",
        num_cores=NUM_CORES, num_subcores=NUM_SUBCORES),
    scratch_types=[
        pltpu.VMEM((BPB,), jnp.int32),
        pltpu.VMEM((BPB,), jnp.int32),
        pltpu.VMEM((NB, IB), jnp.int32),
        pltpu.VMEM((NB, IB), jnp.int32),
        pltpu.SemaphoreType.DMA,
    ],
)


def kernel(mask):
    xt = (mask.astype(jnp.int32)
          .reshape(BATCH, NBLK, LANES, BS)
          .transpose(0, 1, 3, 2)
          .reshape(BATCH * SEQ_LEN))
    out = _sc_call(xt)
    return out[:BATCH * NUMEL].reshape(BATCH, NUMEL)


# speedup vs baseline: 311.3276x; 311.3276x over previous
import jax
import jax.numpy as jnp
from jax import lax
from jax.experimental import pallas as pl
from jax.experimental.pallas import tpu as pltpu
from jax.experimental.pallas import tpu_sc as plsc

BATCH = 128
SEQ_LEN = 32768
NUMEL = 2048

LANES = 16
NUM_CORES = 2
NUM_SUBCORES = 16
NUM_WORKERS = NUM_CORES * NUM_SUBCORES
ROWS_PER_WORKER = BATCH // NUM_WORKERS

BS = 128
BPB = BS * LANES
NBLK = SEQ_LEN // BPB

IB = 128
KPB = IB // LANES
NB = BPB // IB

DPAD = IB
RROWB = NUMEL + DPAD


def _body(mask_hbm, out_hbm, mbuf, rbuf, idxbuf, valbuf, shared, sem):
    sid = lax.axis_index("s")
    wid = sid * NUM_CORES + lax.axis_index("c")
    iota = lax.broadcasted_iota(jnp.int32, (LANES,), 0)
    iota_bs = iota * BS
    iota_kpb = iota * KPB
    zeros16 = jnp.zeros((LANES,), jnp.int32)

    for rl in range(ROWS_PER_WORKER):
        row = wid * ROWS_PER_WORKER + rl
        mrow = pl.multiple_of(row * SEQ_LEN, SEQ_LEN)
        rbase = (sid * ROWS_PER_WORKER + rl) * RROWB
        dump_base = rbase + NUMEL

        def pass_a(k, c):
            vals = mbuf[pl.ds(k * LANES, LANES)]
            rbuf[pl.ds(k * LANES, LANES)] = c
            return c + vals

        def blk_body(blk, t):
            active = t < NUMEL

            @pl.when(active)
            def _load():
                off = pl.multiple_of(mrow + blk * BPB, BPB)
                pltpu.sync_copy(mask_hbm.at[pl.ds(off, BPB)], mbuf)

            n = jnp.where(active, BS, 0)
            c = lax.fori_loop(0, n, pass_a, zeros16)

            bv = zeros16
            run_t = t
            for j in range(LANES):
                bv = jnp.where(iota == j, run_t, bv)
                run_t = run_t + c[j]

            blk_base = blk * BPB

            def pass_b(k, d):
                vals = mbuf[pl.ds(k * LANES, LANES)]
                r = rbuf[pl.ds(k * LANES, LANES)]
                rt = bv + r
                valid = (vals != 0) & (rt < NUMEL)
                dumpv = dump_base + iota_kpb + (k % KPB)
                idxbuf[k // KPB, pl.ds((k % KPB) * LANES, LANES)] = (
                    jnp.where(valid, rbase + rt, dumpv))
                valbuf[k // KPB, pl.ds((k % KPB) * LANES, LANES)] = (
                    blk_base + iota_bs + k)
                return d

            lax.fori_loop(0, n, pass_b, jnp.int32(0))

            @pl.when(active)
            def _scatter():
                for b in range(NB):
                    pltpu.async_copy(valbuf.at[b],
                                     shared.at[idxbuf.at[b]], sem).wait()

            return jnp.where(active, run_t, t)

        t = lax.fori_loop(0, NBLK, blk_body, jnp.int32(0))

        @pl.when(t < NUMEL)
        def _false_sweep():
            need = NUMEL - t

            def blk_body_f(blk, f):
                active = f < need

                @pl.when(active)
                def _load():
                    off = pl.multiple_of(mrow + blk * BPB, BPB)
                    pltpu.sync_copy(mask_hbm.at[pl.ds(off, BPB)], mbuf)

                n = jnp.where(active, BS, 0)
                c = lax.fori_loop(0, n, pass_a, zeros16)

                bfv = zeros16
                run_f = f
                for j in range(LANES):
                    bfv = jnp.where(iota == j, run_f, bfv)
                    run_f = run_f + (BS - c[j])

                blk_base = blk * BPB

                def pass_bf(k, d):
                    vals = mbuf[pl.ds(k * LANES, LANES)]
                    r = rbuf[pl.ds(k * LANES, LANES)]
                    rf = bfv + (k - r)
                    valid = (vals == 0) & (rf < need)
                    dumpv = dump_base + iota_kpb + (k % KPB)
                    idxbuf[k // KPB, pl.ds((k % KPB) * LANES, LANES)] = (
                        jnp.where(valid, rbase + t + rf, dumpv))
                    valbuf[k // KPB, pl.ds((k % KPB) * LANES, LANES)] = (
                        blk_base + iota_bs + k)
                    return d

                lax.fori_loop(0, n, pass_bf, jnp.int32(0))

                @pl.when(active)
                def _scatter():
                    for b in range(NB):
                        pltpu.async_copy(valbuf.at[b],
                                         shared.at[idxbuf.at[b]],
                                         sem).wait()

                return jnp.where(active, run_f, f)

            lax.fori_loop(0, NBLK, blk_body_f, jnp.int32(0))

        pltpu.sync_copy(
            shared.at[pl.ds(pl.multiple_of(rbase, 8), NUMEL)],
            out_hbm.at[pl.ds(pl.multiple_of(row * NUMEL, NUMEL), NUMEL)])


_sc_call = pl.kernel(
    _body,
    out_type=jax.ShapeDtypeStruct((BATCH * NUMEL,), jnp.int32),
    mesh=plsc.VectorSubcoreMesh(
        core_axis_name="c", subcore_axis_name="s",
        num_cores=NUM_CORES, num_subcores=NUM_SUBCORES),
    scratch_types=[
        pltpu.VMEM((BPB,), jnp.int32),
        pltpu.VMEM((BPB,), jnp.int32),
        pltpu.VMEM((NB, IB), jnp.int32),
        pltpu.VMEM((NB, IB), jnp.int32),
        pltpu.VMEM_SHARED((NUM_SUBCORES * ROWS_PER_WORKER * RROWB,),
                          jnp.int32),
        pltpu.SemaphoreType.DMA,
    ],
)


def kernel(mask):
    xt = (mask.astype(jnp.int32)
          .reshape(BATCH, NBLK, LANES, BS)
          .transpose(0, 1, 3, 2)
          .reshape(BATCH * SEQ_LEN))
    return _sc_call(xt).reshape(BATCH, NUMEL)


# speedup vs baseline: 331.1541x; 1.0637x over previous
import jax
import jax.numpy as jnp
from jax import lax
from jax.experimental import pallas as pl
from jax.experimental.pallas import tpu as pltpu
from jax.experimental.pallas import tpu_sc as plsc

BATCH = 128
SEQ_LEN = 32768
NUMEL = 2048

LANES = 16
NUM_CORES = 2
NUM_SUBCORES = 16
NUM_WORKERS = NUM_CORES * NUM_SUBCORES
ROWS_PER_WORKER = BATCH // NUM_WORKERS

BS = 128
BPB = BS * LANES
NBLK = SEQ_LEN // BPB

IB = 128
KPB = IB // LANES
NB = BPB // IB

DPAD = IB
RROWB = NUMEL + DPAD


def _body(mask_hbm, out_hbm, mbuf, rbuf, idxbuf, valbuf, shared, sem):
    sid = lax.axis_index("s")
    wid = sid * NUM_CORES + lax.axis_index("c")
    iota = lax.broadcasted_iota(jnp.int32, (LANES,), 0)
    iota_bs = iota * BS
    iota_kpb = iota * KPB
    zeros16 = jnp.zeros((LANES,), jnp.int32)

    for rl in range(ROWS_PER_WORKER):
        row = wid * ROWS_PER_WORKER + rl
        mrow = pl.multiple_of(row * SEQ_LEN, SEQ_LEN)
        rbase = (sid * ROWS_PER_WORKER + rl) * RROWB
        dump_base = rbase + NUMEL

        def pass_a(k, c):
            vals = mbuf[pl.ds(k * LANES, LANES)]
            rbuf[pl.ds(k * LANES, LANES)] = c
            return c + vals

        def blk_body(blk, t):
            active = t < NUMEL

            @pl.when(active)
            def _load():
                off = pl.multiple_of(mrow + blk * BPB, BPB)
                pltpu.sync_copy(mask_hbm.at[pl.ds(off, BPB)], mbuf)

            n = jnp.where(active, BS, 0)
            c = lax.fori_loop(0, n, pass_a, zeros16)

            bv = zeros16
            run_t = t
            for j in range(LANES):
                bv = jnp.where(iota == j, run_t, bv)
                run_t = run_t + c[j]

            blk_base = blk * BPB

            def pass_b(k, d):
                vals = mbuf[pl.ds(k * LANES, LANES)]
                r = rbuf[pl.ds(k * LANES, LANES)]
                rt = bv + r
                valid = (vals != 0) & (rt < NUMEL)
                dumpv = dump_base + iota_kpb + (k % KPB)
                idxbuf[k // KPB, pl.ds((k % KPB) * LANES, LANES)] = (
                    jnp.where(valid, rbase + rt, dumpv))
                valbuf[k // KPB, pl.ds((k % KPB) * LANES, LANES)] = (
                    blk_base + iota_bs + k)
                return d

            lax.fori_loop(0, n, pass_b, jnp.int32(0))

            @pl.when(active)
            def _scatter():
                copies = [
                    pltpu.async_copy(valbuf.at[b],
                                     shared.at[idxbuf.at[b]], sem)
                    for b in range(NB)
                ]
                for cp in copies:
                    cp.wait()

            return jnp.where(active, run_t, t)

        t = lax.fori_loop(0, NBLK, blk_body, jnp.int32(0))

        @pl.when(t < NUMEL)
        def _false_sweep():
            need = NUMEL - t

            def blk_body_f(blk, f):
                active = f < need

                @pl.when(active)
                def _load():
                    off = pl.multiple_of(mrow + blk * BPB, BPB)
                    pltpu.sync_copy(mask_hbm.at[pl.ds(off, BPB)], mbuf)

                n = jnp.where(active, BS, 0)
                c = lax.fori_loop(0, n, pass_a, zeros16)

                bfv = zeros16
                run_f = f
                for j in range(LANES):
                    bfv = jnp.where(iota == j, run_f, bfv)
                    run_f = run_f + (BS - c[j])

                blk_base = blk * BPB

                def pass_bf(k, d):
                    vals = mbuf[pl.ds(k * LANES, LANES)]
                    r = rbuf[pl.ds(k * LANES, LANES)]
                    rf = bfv + (k - r)
                    valid = (vals == 0) & (rf < need)
                    dumpv = dump_base + iota_kpb + (k % KPB)
                    idxbuf[k // KPB, pl.ds((k % KPB) * LANES, LANES)] = (
                        jnp.where(valid, rbase + t + rf, dumpv))
                    valbuf[k // KPB, pl.ds((k % KPB) * LANES, LANES)] = (
                        blk_base + iota_bs + k)
                    return d

                lax.fori_loop(0, n, pass_bf, jnp.int32(0))

                @pl.when(active)
                def _scatter():
                    copies = [
                        pltpu.async_copy(valbuf.at[b],
                                         shared.at[idxbuf.at[b]], sem)
                        for b in range(NB)
                    ]
                    for cp in copies:
                        cp.wait()

                return jnp.where(active, run_f, f)

            lax.fori_loop(0, NBLK, blk_body_f, jnp.int32(0))

        pltpu.sync_copy(
            shared.at[pl.ds(pl.multiple_of(rbase, 8), NUMEL)],
            out_hbm.at[pl.ds(pl.multiple_of(row * NUMEL, NUMEL), NUMEL)])


_sc_call = pl.kernel(
    _body,
    out_type=jax.ShapeDtypeStruct((BATCH * NUMEL,), jnp.int32),
    mesh=plsc.VectorSubcoreMesh(
        core_axis_name="c", subcore_axis_name="s",
        num_cores=NUM_CORES, num_subcores=NUM_SUBCORES),
    scratch_types=[
        pltpu.VMEM((BPB,), jnp.int32),
        pltpu.VMEM((BPB,), jnp.int32),
        pltpu.VMEM((NB, IB), jnp.int32),
        pltpu.VMEM((NB, IB), jnp.int32),
        pltpu.VMEM_SHARED((NUM_SUBCORES * ROWS_PER_WORKER * RROWB,),
                          jnp.int32),
        pltpu.SemaphoreType.DMA,
    ],
)


def kernel(mask):
    xt = (mask.astype(jnp.int32)
          .reshape(BATCH, NBLK, LANES, BS)
          .transpose(0, 1, 3, 2)
          .reshape(BATCH * SEQ_LEN))
    return _sc_call(xt).reshape(BATCH, NUMEL)


# speedup vs baseline: 547.0210x; 1.6519x over previous
import jax
import jax.numpy as jnp
from jax import lax
from jax.experimental import pallas as pl
from jax.experimental.pallas import tpu as pltpu
from jax.experimental.pallas import tpu_sc as plsc

BATCH = 128
SEQ_LEN = 32768
NUMEL = 2048

LANES = 16
NUM_CORES = 2
NUM_SUBCORES = 16
NUM_WORKERS = NUM_CORES * NUM_SUBCORES
ROWS_PER_WORKER = BATCH // NUM_WORKERS

BS = 128
BPB = BS * LANES
NBLK = SEQ_LEN // BPB

IB = 128
KPB = IB // LANES
NB = BPB // IB

DPAD = IB
RROWB = NUMEL + DPAD


def _body(mask_hbm, out_hbm, mbuf, rbuf, idxbuf, valbuf, shared, sem):
    sid = lax.axis_index("s")
    wid = sid * NUM_CORES + lax.axis_index("c")
    iota = lax.broadcasted_iota(jnp.int32, (LANES,), 0)
    iota_bs = iota * BS
    iota_kpb = iota * KPB
    zeros16 = jnp.zeros((LANES,), jnp.int32)

    for rl in range(ROWS_PER_WORKER):
        row = wid * ROWS_PER_WORKER + rl
        rbase = (sid * ROWS_PER_WORKER + rl) * RROWB
        dump_base = rbase + NUMEL

        def pass_a(k, c):
            vals = mbuf[k, pl.ds(0, LANES)]
            rbuf[pl.ds(k * LANES, LANES)] = c
            return c + vals

        def blk_body(blk, t):
            active = t < NUMEL

            @pl.when(active)
            def _load():
                pltpu.sync_copy(mask_hbm.at[row * NBLK + blk], mbuf)

            n = jnp.where(active, BS, 0)
            c = lax.fori_loop(0, n, pass_a, zeros16)

            bv = zeros16
            run_t = t
            for j in range(LANES):
                bv = jnp.where(iota == j, run_t, bv)
                run_t = run_t + c[j]

            blk_base = blk * BPB

            def pass_b(k, d):
                vals = mbuf[k, pl.ds(0, LANES)]
                r = rbuf[pl.ds(k * LANES, LANES)]
                rt = bv + r
                valid = (vals != 0) & (rt < NUMEL)
                dumpv = dump_base + iota_kpb + (k % KPB)
                idxbuf[k // KPB, pl.ds((k % KPB) * LANES, LANES)] = (
                    jnp.where(valid, rbase + rt, dumpv))
                valbuf[k // KPB, pl.ds((k % KPB) * LANES, LANES)] = (
                    blk_base + iota_bs + k)
                return d

            lax.fori_loop(0, n, pass_b, jnp.int32(0))

            @pl.when(active)
            def _scatter():
                copies = [
                    pltpu.async_copy(valbuf.at[b],
                                     shared.at[idxbuf.at[b]], sem)
                    for b in range(NB)
                ]
                for cp in copies:
                    cp.wait()

            return jnp.where(active, run_t, t)

        t = lax.fori_loop(0, NBLK, blk_body, jnp.int32(0))

        @pl.when(t < NUMEL)
        def _false_sweep():
            need = NUMEL - t

            def blk_body_f(blk, f):
                active = f < need

                @pl.when(active)
                def _load():
                    pltpu.sync_copy(mask_hbm.at[row * NBLK + blk], mbuf)

                n = jnp.where(active, BS, 0)
                c = lax.fori_loop(0, n, pass_a, zeros16)

                bfv = zeros16
                run_f = f
                for j in range(LANES):
                    bfv = jnp.where(iota == j, run_f, bfv)
                    run_f = run_f + (BS - c[j])

                blk_base = blk * BPB

                def pass_bf(k, d):
                    vals = mbuf[k, pl.ds(0, LANES)]
                    r = rbuf[pl.ds(k * LANES, LANES)]
                    rf = bfv + (k - r)
                    valid = (vals == 0) & (rf < need)
                    dumpv = dump_base + iota_kpb + (k % KPB)
                    idxbuf[k // KPB, pl.ds((k % KPB) * LANES, LANES)] = (
                        jnp.where(valid, rbase + t + rf, dumpv))
                    valbuf[k // KPB, pl.ds((k % KPB) * LANES, LANES)] = (
                        blk_base + iota_bs + k)
                    return d

                lax.fori_loop(0, n, pass_bf, jnp.int32(0))

                @pl.when(active)
                def _scatter():
                    copies = [
                        pltpu.async_copy(valbuf.at[b],
                                         shared.at[idxbuf.at[b]], sem)
                        for b in range(NB)
                    ]
                    for cp in copies:
                        cp.wait()

                return jnp.where(active, run_f, f)

            lax.fori_loop(0, NBLK, blk_body_f, jnp.int32(0))

        pltpu.sync_copy(
            shared.at[pl.ds(pl.multiple_of(rbase, 8), NUMEL)],
            out_hbm.at[pl.ds(pl.multiple_of(row * NUMEL, NUMEL), NUMEL)])


_sc_call = pl.kernel(
    _body,
    out_type=jax.ShapeDtypeStruct((BATCH * NUMEL,), jnp.int32),
    mesh=plsc.VectorSubcoreMesh(
        core_axis_name="c", subcore_axis_name="s",
        num_cores=NUM_CORES, num_subcores=NUM_SUBCORES),
    scratch_types=[
        pltpu.VMEM((BS, LANES), jnp.int32),
        pltpu.VMEM((BPB,), jnp.int32),
        pltpu.VMEM((NB, IB), jnp.int32),
        pltpu.VMEM((NB, IB), jnp.int32),
        pltpu.VMEM_SHARED((NUM_SUBCORES * ROWS_PER_WORKER * RROWB,),
                          jnp.int32),
        pltpu.SemaphoreType.DMA,
    ],
)


def kernel(mask):
    xt = (mask.astype(jnp.int32)
          .reshape(BATCH, NBLK, LANES, BS)
          .transpose(0, 1, 3, 2)
          .reshape(BATCH * NBLK, BS, LANES))
    return _sc_call(xt).reshape(BATCH, NUMEL)


# speedup vs baseline: 597.3062x; 1.0919x over previous
import jax
import jax.numpy as jnp
from jax import lax
from jax.experimental import pallas as pl
from jax.experimental.pallas import tpu as pltpu
from jax.experimental.pallas import tpu_sc as plsc

BATCH = 128
SEQ_LEN = 32768
NUMEL = 2048

LANES = 16
NUM_CORES = 2
NUM_SUBCORES = 16
NUM_WORKERS = NUM_CORES * NUM_SUBCORES
ROWS_PER_WORKER = BATCH // NUM_WORKERS

BS = 128
BPB = BS * LANES
NBLK = SEQ_LEN // BPB

IB = 128
KPB = IB // LANES
NB = BPB // IB

DPAD = IB
RROWB = NUMEL + DPAD


def _body(mask_hbm, out_hbm, mbuf, idxbuf, valbuf, shared, sem, sem2):
    sid = lax.axis_index("s")
    wid = sid * NUM_CORES + lax.axis_index("c")
    iota = lax.broadcasted_iota(jnp.int32, (LANES,), 0)
    iota_bs = iota * BS
    iota_kpb = iota * KPB
    zeros16 = jnp.zeros((LANES,), jnp.int32)

    for rl in range(ROWS_PER_WORKER):
        row = wid * ROWS_PER_WORKER + rl
        rbase = (sid * ROWS_PER_WORKER + rl) * RROWB
        dump_base = rbase + NUMEL

        def mk_pass_a(half):
            def pass_a(k, c):
                vals = mbuf[half * BS + k, pl.ds(0, LANES)]
                return c + vals
            return pass_a

        pltpu.sync_copy(mask_hbm.at[row * NBLK], mbuf.at[pl.ds(0, BS)])

        def blk_body(blk, t):
            active = t < NUMEL
            half = blk % 2

            nxt = jnp.minimum(blk + 1, NBLK - 1)
            pf = pltpu.make_async_copy(
                mask_hbm.at[row * NBLK + nxt],
                mbuf.at[pl.ds((1 - half) * BS, BS)], sem2)

            @pl.when(active & (blk < NBLK - 1))
            def _prefetch():
                pf.start()

            n = jnp.where(active, BS, 0)
            c = lax.fori_loop(0, n, mk_pass_a(half), zeros16)

            bv = zeros16
            run_t = t
            for j in range(LANES):
                bv = jnp.where(iota == j, run_t, bv)
                run_t = run_t + c[j]

            blk_base = blk * BPB

            def pass_b(k, r):
                vals = mbuf[half * BS + k, pl.ds(0, LANES)]
                rt = bv + r
                valid = (vals != 0) & (rt < NUMEL)
                dumpv = dump_base + iota_kpb + (k % KPB)
                idxbuf[k // KPB, pl.ds((k % KPB) * LANES, LANES)] = (
                    jnp.where(valid, rbase + rt, dumpv))
                valbuf[k // KPB, pl.ds((k % KPB) * LANES, LANES)] = (
                    blk_base + iota_bs + k)
                return r + vals

            lax.fori_loop(0, n, pass_b, zeros16)

            @pl.when(active)
            def _scatter():
                copies = [
                    pltpu.async_copy(valbuf.at[b],
                                     shared.at[idxbuf.at[b]], sem)
                    for b in range(NB)
                ]
                for cp in copies:
                    cp.wait()

            @pl.when(active & (blk < NBLK - 1))
            def _pf_wait():
                pf.wait()

            return jnp.where(active, run_t, t)

        t = lax.fori_loop(0, NBLK, blk_body, jnp.int32(0))

        @pl.when(t < NUMEL)
        def _false_sweep():
            need = NUMEL - t

            def blk_body_f(blk, f):
                active = f < need

                @pl.when(active)
                def _load():
                    pltpu.sync_copy(mask_hbm.at[row * NBLK + blk],
                                    mbuf.at[pl.ds(0, BS)])

                n = jnp.where(active, BS, 0)
                c = lax.fori_loop(0, n, mk_pass_a(0), zeros16)

                bfv = zeros16
                run_f = f
                for j in range(LANES):
                    bfv = jnp.where(iota == j, run_f, bfv)
                    run_f = run_f + (BS - c[j])

                blk_base = blk * BPB

                def pass_bf(k, r):
                    vals = mbuf[k, pl.ds(0, LANES)]
                    rf = bfv + (k - r)
                    valid = (vals == 0) & (rf < need)
                    dumpv = dump_base + iota_kpb + (k % KPB)
                    idxbuf[k // KPB, pl.ds((k % KPB) * LANES, LANES)] = (
                        jnp.where(valid, rbase + t + rf, dumpv))
                    valbuf[k // KPB, pl.ds((k % KPB) * LANES, LANES)] = (
                        blk_base + iota_bs + k)
                    return r + vals

                lax.fori_loop(0, n, pass_bf, zeros16)

                @pl.when(active)
                def _scatter():
                    copies = [
                        pltpu.async_copy(valbuf.at[b],
                                         shared.at[idxbuf.at[b]], sem)
                        for b in range(NB)
                    ]
                    for cp in copies:
                        cp.wait()

                return jnp.where(active, run_f, f)

            lax.fori_loop(0, NBLK, blk_body_f, jnp.int32(0))

        pltpu.sync_copy(
            shared.at[pl.ds(pl.multiple_of(rbase, 8), NUMEL)],
            out_hbm.at[pl.ds(pl.multiple_of(row * NUMEL, NUMEL), NUMEL)])


_sc_call = pl.kernel(
    _body,
    out_type=jax.ShapeDtypeStruct((BATCH * NUMEL,), jnp.int32),
    mesh=plsc.VectorSubcoreMesh(
        core_axis_name="c", subcore_axis_name="s",
        num_cores=NUM_CORES, num_subcores=NUM_SUBCORES),
    scratch_types=[
        pltpu.VMEM((2 * BS, LANES), jnp.int32),
        pltpu.VMEM((NB, IB), jnp.int32),
        pltpu.VMEM((NB, IB), jnp.int32),
        pltpu.VMEM_SHARED((NUM_SUBCORES * ROWS_PER_WORKER * RROWB,),
                          jnp.int32),
        pltpu.SemaphoreType.DMA,
        pltpu.SemaphoreType.DMA,
    ],
)


def kernel(mask):
    xt = (mask.astype(jnp.int32)
          .reshape(BATCH, NBLK, LANES, BS)
          .transpose(0, 1, 3, 2)
          .reshape(BATCH * NBLK, BS, LANES))
    return _sc_call(xt).reshape(BATCH, NUMEL)


# speedup vs baseline: 693.0599x; 1.1603x over previous
import jax
import jax.numpy as jnp
from jax import lax
from jax.experimental import pallas as pl
from jax.experimental.pallas import tpu as pltpu
from jax.experimental.pallas import tpu_sc as plsc

BATCH = 128
SEQ_LEN = 32768
NUMEL = 2048

LANES = 16
NUM_CORES = 2
NUM_SUBCORES = 16
NUM_WORKERS = NUM_CORES * NUM_SUBCORES
ROWS_PER_WORKER = BATCH // NUM_WORKERS

BS = 128
BPB = BS * LANES
NBLK = SEQ_LEN // BPB

IB = 128
KPB = IB // LANES
NB = BPB // IB
KG = BS // 4

DPAD = IB
RROWB = NUMEL + DPAD


def _body(mask_hbm, out_hbm, mbuf, idxbuf, valbuf, shared, sem, sem2):
    sid = lax.axis_index("s")
    wid = sid * NUM_CORES + lax.axis_index("c")
    iota = lax.broadcasted_iota(jnp.int32, (LANES,), 0)
    iota_bs = iota * BS
    iota_kpb = iota * KPB
    zeros16 = jnp.zeros((LANES,), jnp.int32)

    for rl in range(ROWS_PER_WORKER):
        row = wid * ROWS_PER_WORKER + rl
        rbase = (sid * ROWS_PER_WORKER + rl) * RROWB
        dump_base = rbase + NUMEL

        def mk_pass_a(half):
            def pass_a(kw, c):
                w = mbuf[half * KG + kw, pl.ds(0, LANES)]
                c = c + (w & 1) + ((w >> 8) & 1)
                return c + ((w >> 16) & 1) + ((w >> 24) & 1)
            return pass_a

        pltpu.sync_copy(mask_hbm.at[row * NBLK], mbuf.at[pl.ds(0, KG)])

        def blk_body(blk, t):
            active = t < NUMEL
            half = blk % 2

            nxt = jnp.minimum(blk + 1, NBLK - 1)
            pf = pltpu.make_async_copy(
                mask_hbm.at[row * NBLK + nxt],
                mbuf.at[pl.ds((1 - half) * KG, KG)], sem2)

            @pl.when(active & (blk < NBLK - 1))
            def _prefetch():
                pf.start()

            n = jnp.where(active, KG, 0)
            c = lax.fori_loop(0, n, mk_pass_a(half), zeros16)

            bv = zeros16
            run_t = t
            for j in range(LANES):
                bv = jnp.where(iota == j, run_t, bv)
                run_t = run_t + c[j]

            blk_base = blk * BPB

            def pass_b(kw, r):
                w = mbuf[half * KG + kw, pl.ds(0, LANES)]
                for ci in range(4):
                    b_ = (w >> (8 * ci)) & 1
                    k = kw * 4 + ci
                    rt = bv + r
                    valid = (b_ != 0) & (rt < NUMEL)
                    dumpv = dump_base + iota_kpb + (k % KPB)
                    idxbuf[k // KPB, pl.ds((k % KPB) * LANES, LANES)] = (
                        jnp.where(valid, rbase + rt, dumpv))
                    valbuf[k // KPB, pl.ds((k % KPB) * LANES, LANES)] = (
                        blk_base + iota_bs + k)
                    r = r + b_
                return r

            lax.fori_loop(0, n, pass_b, zeros16)

            @pl.when(active)
            def _scatter():
                copies = [
                    pltpu.async_copy(valbuf.at[b],
                                     shared.at[idxbuf.at[b]], sem)
                    for b in range(NB)
                ]
                for cp in copies:
                    cp.wait()

            @pl.when(active & (blk < NBLK - 1))
            def _pf_wait():
                pf.wait()

            return jnp.where(active, run_t, t)

        t = lax.fori_loop(0, NBLK, blk_body, jnp.int32(0))

        @pl.when(t < NUMEL)
        def _false_sweep():
            need = NUMEL - t

            def blk_body_f(blk, f):
                active = f < need

                @pl.when(active)
                def _load():
                    pltpu.sync_copy(mask_hbm.at[row * NBLK + blk],
                                    mbuf.at[pl.ds(0, KG)])

                n = jnp.where(active, KG, 0)
                c = lax.fori_loop(0, n, mk_pass_a(0), zeros16)

                bfv = zeros16
                run_f = f
                for j in range(LANES):
                    bfv = jnp.where(iota == j, run_f, bfv)
                    run_f = run_f + (BS - c[j])

                blk_base = blk * BPB

                def pass_bf(kw, r):
                    w = mbuf[kw, pl.ds(0, LANES)]
                    for ci in range(4):
                        b_ = (w >> (8 * ci)) & 1
                        k = kw * 4 + ci
                        rf = bfv + (k - r)
                        valid = (b_ == 0) & (rf < need)
                        dumpv = dump_base + iota_kpb + (k % KPB)
                        idxbuf[k // KPB, pl.ds((k % KPB) * LANES, LANES)] = (
                            jnp.where(valid, rbase + t + rf, dumpv))
                        valbuf[k // KPB,
                               pl.ds((k % KPB) * LANES, LANES)] = (
                            blk_base + iota_bs + k)
                        r = r + b_
                    return r

                lax.fori_loop(0, n, pass_bf, zeros16)

                @pl.when(active)
                def _scatter():
                    copies = [
                        pltpu.async_copy(valbuf.at[b],
                                         shared.at[idxbuf.at[b]], sem)
                        for b in range(NB)
                    ]
                    for cp in copies:
                        cp.wait()

                return jnp.where(active, run_f, f)

            lax.fori_loop(0, NBLK, blk_body_f, jnp.int32(0))

        pltpu.sync_copy(
            shared.at[pl.ds(pl.multiple_of(rbase, 8), NUMEL)],
            out_hbm.at[pl.ds(pl.multiple_of(row * NUMEL, NUMEL), NUMEL)])


_sc_call = pl.kernel(
    _body,
    out_type=jax.ShapeDtypeStruct((BATCH * NUMEL,), jnp.int32),
    mesh=plsc.VectorSubcoreMesh(
        core_axis_name="c", subcore_axis_name="s",
        num_cores=NUM_CORES, num_subcores=NUM_SUBCORES),
    scratch_types=[
        pltpu.VMEM((2 * KG, LANES), jnp.int32),
        pltpu.VMEM((NB, IB), jnp.int32),
        pltpu.VMEM((NB, IB), jnp.int32),
        pltpu.VMEM_SHARED((NUM_SUBCORES * ROWS_PER_WORKER * RROWB,),
                          jnp.int32),
        pltpu.SemaphoreType.DMA,
        pltpu.SemaphoreType.DMA,
    ],
)


def kernel(mask):
    x8 = (mask.astype(jnp.uint8)
          .reshape(BATCH, NBLK, LANES, KG, 4)
          .transpose(0, 1, 3, 2, 4))
    xt = lax.bitcast_convert_type(x8, jnp.int32).reshape(
        BATCH * NBLK, KG, LANES)
    return _sc_call(xt).reshape(BATCH, NUMEL)
